# baseline (speedup 1.0000x reference)
import functools

import jax
import jax.numpy as jnp
from jax import lax
from jax.experimental import pallas as pl
from jax.experimental.pallas import tpu as pltpu
from jax.experimental.pallas import tpu_sc as plsc

_V = 601
_VP = 640
_C = 27
_CP = 32
_B = 16384
_RPW = _VP // 16


@functools.cache
def _make_kernel():
    info = plsc.get_sparse_core_info()
    num_workers = info.num_cores * info.num_subcores
    bpw = _B // num_workers
    mesh = plsc.VectorSubcoreMesh(core_axis_name="c", subcore_axis_name="s")

    @functools.partial(
        pl.kernel,
        mesh=mesh,
        out_type=jax.ShapeDtypeStruct((_B, _CP), jnp.float32),
        scratch_types=[
            pltpu.VMEM((_RPW * _C + 16,), jnp.float32),
            pltpu.VMEM((_RPW, _CP), jnp.float32),
            pltpu.VMEM_SHARED((_VP, _CP), jnp.float32),
            pltpu.VMEM((bpw,), jnp.int32),
            pltpu.VMEM((bpw, _CP), jnp.float32),
            pltpu.SemaphoreType.DMA,
        ],
        compiler_params=pltpu.CompilerParams(
            use_tc_tiling_on_sc=False, needs_layout_passes=False,
            skip_device_barrier=True),
    )
    def fused_kernel(w_hbm, idx_hbm, out_hbm, w_v, p_v, table_sh, idx_v,
                     rows_v, sem):
        cid = lax.axis_index("c")
        sid = lax.axis_index("s")
        wid = sid * info.num_cores + cid

        pltpu.sync_copy(
            w_hbm.at[pl.ds(sid * (_RPW * _C), _RPW * _C)],
            w_v.at[pl.ds(0, _RPW * _C)],
        )
        pltpu.sync_copy(idx_hbm.at[pl.ds(wid * bpw, bpw)], idx_v)

        lane = lax.broadcasted_iota(jnp.int32, (16,), 0)
        tail_mask = lane < (_C - 16)
        for r in range(_RPW):
            x1 = w_v[pl.ds(r * _C, 16)]
            x2 = w_v[pl.ds(r * _C + 16, 16)]
            e1 = jnp.exp(x1)
            e2 = jnp.where(tail_mask, jnp.exp(x2), 0.0)
            total = jnp.sum(e1) + jnp.sum(e2)
            row = jnp.full((16,), r, jnp.int32)
            plsc.store_scatter(p_v, [row, lane], e1 / total)
            plsc.store_scatter(p_v, [row, lane + 16], e2 / total,
                               mask=tail_mask)

        pltpu.sync_copy(p_v, table_sh.at[pl.ds(sid * _RPW, _RPW)])
        plsc.subcore_barrier()

        pltpu.async_copy(table_sh.at[idx_v], rows_v, sem).wait()
        pltpu.sync_copy(rows_v, out_hbm.at[pl.ds(wid * bpw, bpw)])

    return fused_kernel


@jax.jit
def kernel(bigram_idx, W):
    w_flat = jnp.pad(W.reshape(-1), (0, _VP * _C - _V * _C))
    out = _make_kernel()(w_flat, bigram_idx.astype(jnp.int32))
    return out[:, :_C]


# speedup vs baseline: 1.5512x; 1.5512x over previous
import functools

import jax
import jax.numpy as jnp
from jax import lax
from jax.experimental import pallas as pl
from jax.experimental.pallas import tpu as pltpu

_V = 601
_C = 27
_B = 16384
_BLK = 4096
_STEPS = _B // _BLK


def _body(idx_ref, w_ref, out_ref, tab_ref):
    @pl.when(pl.program_id(0) == 0)
    def _():
        x = w_ref[...]
        e = jnp.exp(x)
        s = jnp.sum(e, axis=1, keepdims=True)
        tab_ref[...] = (e / s).astype(jnp.bfloat16)

    idx = idx_ref[...]
    rows = lax.broadcasted_iota(jnp.int32, (_BLK, _V), 1)
    onehot = (idx == rows).astype(jnp.bfloat16)
    out_ref[...] = jnp.dot(onehot, tab_ref[...],
                           preferred_element_type=jnp.float32)


_lookup = pl.pallas_call(
    _body,
    grid=(_STEPS,),
    in_specs=[
        pl.BlockSpec((_BLK, 1), lambda i: (i, 0)),
        pl.BlockSpec((_V, _C), lambda i: (0, 0)),
    ],
    out_specs=pl.BlockSpec((_BLK, _C), lambda i: (i, 0)),
    out_shape=jax.ShapeDtypeStruct((_B, _C), jnp.float32),
    scratch_shapes=[pltpu.VMEM((_V, _C), jnp.bfloat16)],
    compiler_params=pltpu.CompilerParams(
        dimension_semantics=("arbitrary",)),
)


@jax.jit
def kernel(bigram_idx, W):
    idx2 = bigram_idx.astype(jnp.int32).reshape(_B, 1)
    return _lookup(idx2, W)


# speedup vs baseline: 2.2090x; 1.4241x over previous
import functools

import jax
import jax.numpy as jnp
from jax import lax
from jax.experimental import pallas as pl
from jax.experimental.pallas import tpu as pltpu

_V = 601
_C = 27
_B = 16384
_BLK = 4096
_STEPS = _B // _BLK


def _body(idx_ref, w_ref, out_ref, tab_ref):
    @pl.when(pl.program_id(0) == 0)
    def _():
        x = w_ref[...]
        e = jnp.exp(x)
        s = jnp.sum(e, axis=1, keepdims=True)
        tab_ref[...] = (e / s).astype(jnp.bfloat16)

    idx = idx_ref[...]
    rows = lax.broadcasted_iota(jnp.int32, (_BLK, _V), 1)
    onehot = (idx[:, None] == rows).astype(jnp.bfloat16)
    out_ref[...] = jnp.dot(onehot, tab_ref[...],
                           preferred_element_type=jnp.float32)


_lookup = pl.pallas_call(
    _body,
    grid=(_STEPS,),
    in_specs=[
        pl.BlockSpec((_BLK,), lambda i: (i,)),
        pl.BlockSpec((_V, _C), lambda i: (0, 0)),
    ],
    out_specs=pl.BlockSpec((_BLK, _C), lambda i: (i, 0)),
    out_shape=jax.ShapeDtypeStruct((_B, _C), jnp.float32),
    scratch_shapes=[pltpu.VMEM((_V, _C), jnp.bfloat16)],
    compiler_params=pltpu.CompilerParams(
        dimension_semantics=("arbitrary",)),
)


@jax.jit
def kernel(bigram_idx, W):
    return _lookup(bigram_idx.astype(jnp.int32), W)
